# Initial kernel scaffold; baseline (speedup 1.0000x reference)
import jax
import jax.numpy as jnp
from jax.experimental import pallas as pl


def kernel(x, edge_index, batch, bn_feat_g, bn_feat_b, bn0_g, bn0_b, W0, b0, bn1_g, bn1_b, W1, b1, bn2_g, bn2_b, W2, b2, bn3_g, bn3_b, W3, b3, bnfc_g, bnfc_b, Wfc, bfc, bnh_g, bnh_b, Wcls, bcls):
    raise NotImplementedError("write your pallas kernel here")


# speedup vs baseline: 1.6421x; 1.6421x over previous
import functools

import jax
import jax.numpy as jnp
from jax import lax
from jax.experimental import pallas as pl
from jax.experimental.pallas import tpu as pltpu
from jax.experimental.pallas import tpu_sc as plsc

N = 10000
E = 160000
IN_DIM = 256
HID = 512
OUT = 128
GRAPHS = 64
EPS = 1e-5
SINV = float(1.0 / (1.0 + EPS) ** 0.5)

NCORE = 2
NSUB = 16
CH = 128
NCHUNK = HID // CH
CPC = NCHUNK // NCORE
EPAD = 163840
ES = EPAD // NSUB
NJ = ES // CH
NR = 3
THIRD = 3456
NROWS = NR * THIRD
NPADT = THIRD + 8
WPS = THIRD // NSUB
NB = 1000
GRID = N // NB

_F32 = jnp.float32


def _sc_mesh():
    return plsc.VectorSubcoreMesh(core_axis_name="c", subcore_axis_name="s",
                                  num_cores=NCORE, num_subcores=NSUB)


def _sc_degree(col_degr, ones16, zeros16):
    @functools.partial(
        pl.kernel,
        out_type=jax.ShapeDtypeStruct((NCORE, NROWS, CH), _F32),
        mesh=_sc_mesh(),
        scratch_types=[
            pltpu.VMEM((40, CH), jnp.int32),
            pltpu.VMEM((CH, CH), _F32),
            pltpu.VMEM((WPS, CH), _F32),
            pltpu.VMEM_SHARED((NPADT, CH), _F32),
        ],
    )
    def deg_kernel(col_hbm, ones_hbm, zr_hbm, deg_hbm, colv, onesv, zerov, acc):
        c = lax.axis_index("c")
        s = lax.axis_index("s")
        pltpu.sync_copy(ones_hbm, onesv)
        pltpu.sync_copy(zr_hbm, zerov)
        for r in range(NR):
            pltpu.sync_copy(zerov, acc.at[pl.ds(s * WPS, WPS)])

            @pl.when(s == 0)
            def _():
                pltpu.sync_copy(zerov.at[pl.ds(0, 8)], acc.at[pl.ds(THIRD, 8)])

            pltpu.sync_copy(col_hbm.at[r, c, s], colv)
            plsc.subcore_barrier()

            @pl.loop(0, 40)
            def _(j):
                pltpu.sync_copy(onesv, acc.at[colv.at[j]], add=True)

            plsc.subcore_barrier()
            pltpu.sync_copy(acc.at[pl.ds(s * WPS, WPS)],
                            deg_hbm.at[c, pl.ds(r * THIRD + s * WPS, WPS)])
            plsc.subcore_barrier()

    return deg_kernel(col_degr, ones16, zeros16)


def _sc_aggregate(u_flat, row3, col3r, zeros):
    @functools.partial(
        pl.kernel,
        out_type=jax.ShapeDtypeStruct((NCHUNK, NROWS, CH), _F32),
        mesh=_sc_mesh(),
        scratch_types=[
            pltpu.VMEM((NJ, CH), jnp.int32),
            pltpu.VMEM((NJ, CH), jnp.int32),
            pltpu.VMEM((2, CH, CH), _F32),
            pltpu.VMEM((WPS, CH), _F32),
            pltpu.VMEM_SHARED((NPADT, CH), _F32),
            pltpu.SemaphoreType.DMA,
            pltpu.SemaphoreType.DMA,
        ],
    )
    def agg_kernel(u_hbm, row_hbm, col_hbm, zr_hbm, raw_hbm,
                   rowv, colv, gv, zv, acc, sem0, sem1):
        c = lax.axis_index("c")
        s = lax.axis_index("s")
        pltpu.sync_copy(zr_hbm, zv)

        def pass_body(p, r):
            f = c * CPC + p
            pltpu.sync_copy(zv, acc.at[pl.ds(s * WPS, WPS)])

            @pl.when(s == 0)
            def _():
                pltpu.sync_copy(zv.at[pl.ds(0, 8)], acc.at[pl.ds(THIRD, 8)])

            pltpu.sync_copy(col_hbm.at[r, s], colv)
            pltpu.sync_copy(row_hbm.at[f, s], rowv)
            plsc.subcore_barrier()

            @pl.loop(0, NJ, step=2)
            def _(j):
                cp0 = pltpu.make_async_copy(u_hbm.at[rowv.at[j]], gv.at[0], sem0)
                cp0.start()
                cp1 = pltpu.make_async_copy(u_hbm.at[rowv.at[j + 1]], gv.at[1], sem1)
                cp1.start()
                cp0.wait()
                pltpu.sync_copy(gv.at[0], acc.at[colv.at[j]], add=True)
                cp1.wait()
                pltpu.sync_copy(gv.at[1], acc.at[colv.at[j + 1]], add=True)

            plsc.subcore_barrier()
            pltpu.sync_copy(acc.at[pl.ds(s * WPS, WPS)],
                            raw_hbm.at[f, pl.ds(r * THIRD + s * WPS, WPS)])
            plsc.subcore_barrier()

        for p in range(CPC):
            for r in range(NR):
                pass_body(p, r)

    return agg_kernel(u_flat, row3, col3r, zeros)


def _vec_spec(d):
    return pl.BlockSpec((1, d), lambda i: (0, 0))


def _chunked(un_ref, un):
    for k in range(NCHUNK):
        un_ref[k] = un[:, k * CH:(k + 1) * CH]


def _tc_prep(x, degp, fg, fb, g0, b0, W0):
    def body(x_ref, deg_ref, fg_ref, fb_ref, g0_ref, b0_ref, w_ref,
             u0_ref, dinv_ref):
        d = deg_ref[0, :, 0:1] + deg_ref[1, :, 0:1] + 1.0
        dinv = lax.rsqrt(jnp.maximum(d, 1.0))
        t0 = g0_ref[...] * ((fg_ref[...] * (x_ref[...] * SINV) + fb_ref[...]) * SINV) + b0_ref[...]
        z0 = jnp.dot(t0, w_ref[...], preferred_element_type=_F32,
                     precision=lax.Precision.HIGHEST)
        _chunked(u0_ref, z0 * dinv)
        dinv_ref[...] = dinv

    return pl.pallas_call(
        body,
        grid=(GRID,),
        in_specs=[
            pl.BlockSpec((NB, IN_DIM), lambda i: (i, 0)),
            pl.BlockSpec((NCORE, NB, CH), lambda i: (0, i, 0)),
            _vec_spec(IN_DIM), _vec_spec(IN_DIM), _vec_spec(IN_DIM), _vec_spec(IN_DIM),
            pl.BlockSpec((IN_DIM, HID), lambda i: (0, 0)),
        ],
        out_specs=[
            pl.BlockSpec((NCHUNK, NB, CH), lambda i: (0, i, 0)),
            pl.BlockSpec((NB, 1), lambda i: (i, 0)),
        ],
        out_shape=[
            jax.ShapeDtypeStruct((NCHUNK, N, CH), _F32),
            jax.ShapeDtypeStruct((N, 1), _F32),
        ],
    )(x, degp, fg, fb, g0, b0, W0)


def _tc_mid(raw, u, hprev, dinv, bias_prev, W, bng, bnb):
    def body(raw_ref, u_ref, hp_ref, dinv_ref, bp_ref, w_ref, g_ref, bb_ref,
             h_ref, un_ref):
        dinv = dinv_ref[...]
        z = jnp.concatenate(
            [dinv * (raw_ref[k] + u_ref[k]) for k in range(NCHUNK)], axis=1)
        h = jnp.maximum(z + bp_ref[...], 0.0) + hp_ref[...]
        h_ref[...] = h
        t = g_ref[...] * (h * SINV) + bb_ref[...]
        zn = jnp.dot(t, w_ref[...], preferred_element_type=_F32,
                     precision=lax.Precision.HIGHEST)
        _chunked(un_ref, zn * dinv)

    return pl.pallas_call(
        body,
        grid=(GRID,),
        in_specs=[
            pl.BlockSpec((NCHUNK, NB, CH), lambda i: (0, i, 0)),
            pl.BlockSpec((NCHUNK, NB, CH), lambda i: (0, i, 0)),
            pl.BlockSpec((NB, HID), lambda i: (i, 0)),
            pl.BlockSpec((NB, 1), lambda i: (i, 0)),
            _vec_spec(HID),
            pl.BlockSpec((HID, HID), lambda i: (0, 0)),
            _vec_spec(HID), _vec_spec(HID),
        ],
        out_specs=[
            pl.BlockSpec((NB, HID), lambda i: (i, 0)),
            pl.BlockSpec((NCHUNK, NB, CH), lambda i: (0, i, 0)),
        ],
        out_shape=[
            jax.ShapeDtypeStruct((N, HID), _F32),
            jax.ShapeDtypeStruct((NCHUNK, N, CH), _F32),
        ],
    )(raw, u, hprev, dinv, bias_prev, W, bng, bnb)


def _tc_last(h4, batch2, fcg, fcb, Wfc, bfc, bhg, bhb, Wcls, bcls):
    def body(h_ref, batch_ref,
             fcg_ref, fcb_ref, wfc_ref, bfc_ref, bhg_ref, bhb_ref,
             wcls_ref, bcls_ref, out_ref, pool_acc, cnt_acc):
        i = pl.program_id(0)

        @pl.when(i == 0)
        def _():
            pool_acc[...] = jnp.zeros_like(pool_acc)
            cnt_acc[...] = jnp.zeros_like(cnt_acc)

        h = h_ref[...]
        gids = lax.broadcasted_iota(jnp.int32, (1, GRAPHS), 1)
        onehot = (batch_ref[...] == gids).astype(_F32)
        dn = (((0,), (0,)), ((), ()))
        pool_acc[...] += lax.dot_general(onehot, h, dn,
                                         preferred_element_type=_F32,
                                         precision=lax.Precision.HIGHEST)
        cnt_acc[...] += lax.dot_general(onehot, jnp.ones((NB, CH), _F32), dn,
                                        preferred_element_type=_F32,
                                        precision=lax.Precision.HIGHEST)

        @pl.when(i == GRID - 1)
        def _():
            cnt = cnt_acc[:, 0:1]
            hp = pool_acc[...] / jnp.maximum(cnt, 1.0)
            h_ = fcg_ref[...] * (hp * SINV) + fcb_ref[...]
            h_ = jnp.dot(h_, wfc_ref[...], preferred_element_type=_F32,
                         precision=lax.Precision.HIGHEST) + bfc_ref[...]
            hp = hp + jnp.maximum(h_, 0.0)
            hh = bhg_ref[...] * (hp * SINV) + bhb_ref[...]
            o = jnp.dot(hh, wcls_ref[...], preferred_element_type=_F32,
                        precision=lax.Precision.HIGHEST) + bcls_ref[...]
            m = jnp.max(o, axis=1, keepdims=True)
            lse = m + jnp.log(jnp.sum(jnp.exp(o - m), axis=1, keepdims=True))
            out_ref[...] = o - lse

    return pl.pallas_call(
        body,
        grid=(GRID,),
        in_specs=[
            pl.BlockSpec((NB, HID), lambda i: (i, 0)),
            pl.BlockSpec((NB, 1), lambda i: (i, 0)),
            _vec_spec(HID), _vec_spec(HID),
            pl.BlockSpec((HID, HID), lambda i: (0, 0)),
            _vec_spec(HID), _vec_spec(HID), _vec_spec(HID),
            pl.BlockSpec((HID, OUT), lambda i: (0, 0)),
            _vec_spec(OUT),
        ],
        out_specs=pl.BlockSpec((GRAPHS, OUT), lambda i: (0, 0)),
        out_shape=jax.ShapeDtypeStruct((GRAPHS, OUT), _F32),
        scratch_shapes=[
            pltpu.VMEM((GRAPHS, HID), _F32),
            pltpu.VMEM((GRAPHS, CH), _F32),
        ],
    )(h4, batch2, fcg, fcb, Wfc, bfc, bhg, bhb, Wcls, bcls)


def kernel(x, edge_index, batch, bn_feat_g, bn_feat_b,
           bn0_g, bn0_b, W0, b0, bn1_g, bn1_b, W1, b1,
           bn2_g, bn2_b, W2, b2, bn3_g, bn3_b, W3, b3,
           bnfc_g, bnfc_b, Wfc, bfc, bnh_g, bnh_b, Wcls, bcls):
    v = lambda a: a.reshape(1, -1)
    row = edge_index[0].astype(jnp.int32)
    col = edge_index[1].astype(jnp.int32)
    pad = EPAD - E
    rowp = jnp.concatenate([row, jnp.zeros((pad,), jnp.int32)])
    colp = jnp.concatenate([col, jnp.full((pad,), N, jnp.int32)])
    chs = [jnp.where((colp >= r * THIRD) & (colp < (r + 1) * THIRD),
                     colp - r * THIRD, THIRD) for r in range(NR)]
    col3r = jnp.stack(chs).reshape(NR, NSUB, NJ, CH)
    col_degr = jnp.stack(chs).reshape(
        NR, NCORE, NSUB, EPAD // (NCORE * NSUB * CH), CH)
    row3 = (rowp[None, :] + (jnp.arange(NCHUNK, dtype=jnp.int32) * N)[:, None]
            ).reshape(NCHUNK, NSUB, NJ, CH)
    zerosw = jnp.zeros((WPS, CH), _F32)
    onesw = jnp.ones((CH, CH), _F32)

    degp = _sc_degree(col_degr, onesw, zerosw)
    u0, dinv = _tc_prep(x, degp, v(bn_feat_g), v(bn_feat_b), v(bn0_g), v(bn0_b), W0)

    wstack = jnp.stack([W1, W2, W3, W3])
    bstack = jnp.stack([v(b0), v(b1), v(b2), v(b3)])
    gstack = jnp.stack([v(bn1_g), v(bn2_g), v(bn3_g), v(bn3_g)])
    bbstack = jnp.stack([v(bn1_b), v(bn2_b), v(bn3_b), v(bn3_b)])

    n_it = jnp.where(jnp.max(batch) > jnp.int32(1 << 30), 5, 4).astype(jnp.int32)

    def stage(state):
        i, hprev, u = state
        raw = _sc_aggregate(u.reshape(NCHUNK * N, CH), row3, col3r, zerosw)
        Wn = lax.dynamic_index_in_dim(wstack, i, keepdims=False)
        bp = lax.dynamic_index_in_dim(bstack, i, keepdims=False)
        gn = lax.dynamic_index_in_dim(gstack, i, keepdims=False)
        bbn = lax.dynamic_index_in_dim(bbstack, i, keepdims=False)
        h, u_next = _tc_mid(raw, u, hprev, dinv, bp, Wn, gn, bbn)
        return (i + 1, h, u_next)

    _, h4, _ = lax.while_loop(lambda st: st[0] < n_it, stage,
                              (jnp.int32(0), jnp.zeros((N, HID), _F32), u0))
    return _tc_last(h4, batch.reshape(N, 1),
                    v(bnfc_g), v(bnfc_b), Wfc, v(bfc),
                    v(bnh_g), v(bnh_b), Wcls, v(bcls))


# speedup vs baseline: 1.7518x; 1.0668x over previous
import functools

import jax
import jax.numpy as jnp
from jax import lax
from jax.experimental import pallas as pl
from jax.experimental.pallas import tpu as pltpu
from jax.experimental.pallas import tpu_sc as plsc

N = 10000
E = 160000
IN_DIM = 256
HID = 512
OUT = 128
GRAPHS = 64
EPS = 1e-5
SINV = float(1.0 / (1.0 + EPS) ** 0.5)

NCORE = 2
NSUB = 16
CH = 128
NCHUNK = HID // CH
CPC = NCHUNK // NCORE
EPAD = 163840
ES = EPAD // NSUB
NJ = ES // CH
NR = 3
THIRD = 3456
NROWS = NR * THIRD
NPADT = THIRD + 8
WPS = THIRD // NSUB
NB = 1000
GRID = N // NB

_F32 = jnp.float32


def _sc_mesh():
    return plsc.VectorSubcoreMesh(core_axis_name="c", subcore_axis_name="s",
                                  num_cores=NCORE, num_subcores=NSUB)


def _sc_degree(col_degr, ones16, zeros16):
    @functools.partial(
        pl.kernel,
        out_type=jax.ShapeDtypeStruct((NCORE, NROWS, CH), _F32),
        mesh=_sc_mesh(),
        scratch_types=[
            pltpu.VMEM((40, CH), jnp.int32),
            pltpu.VMEM((CH, CH), _F32),
            pltpu.VMEM((WPS, CH), _F32),
            pltpu.VMEM_SHARED((NPADT, CH), _F32),
        ],
    )
    def deg_kernel(col_hbm, ones_hbm, zr_hbm, deg_hbm, colv, onesv, zerov, acc):
        c = lax.axis_index("c")
        s = lax.axis_index("s")
        pltpu.sync_copy(ones_hbm, onesv)
        pltpu.sync_copy(zr_hbm, zerov)
        for r in range(NR):
            pltpu.sync_copy(zerov, acc.at[pl.ds(s * WPS, WPS)])

            @pl.when(s == 0)
            def _():
                pltpu.sync_copy(zerov.at[pl.ds(0, 8)], acc.at[pl.ds(THIRD, 8)])

            pltpu.sync_copy(col_hbm.at[r, c, s], colv)
            plsc.subcore_barrier()

            @pl.loop(0, 40)
            def _(j):
                pltpu.sync_copy(onesv, acc.at[colv.at[j]], add=True)

            plsc.subcore_barrier()
            pltpu.sync_copy(acc.at[pl.ds(s * WPS, WPS)],
                            deg_hbm.at[c, pl.ds(r * THIRD + s * WPS, WPS)])
            plsc.subcore_barrier()

    return deg_kernel(col_degr, ones16, zeros16)


def _sc_aggregate(u_flat, row3, col3r, zeros):
    @functools.partial(
        pl.kernel,
        out_type=jax.ShapeDtypeStruct((NCHUNK, NROWS, CH), _F32),
        mesh=_sc_mesh(),
        scratch_types=[
            pltpu.VMEM((NJ, CH), jnp.int32),
            pltpu.VMEM((NJ, CH), jnp.int32),
            pltpu.VMEM((2, CH, CH), _F32),
            pltpu.VMEM((WPS, CH), _F32),
            pltpu.VMEM_SHARED((NPADT, CH), _F32),
            pltpu.SemaphoreType.DMA,
            pltpu.SemaphoreType.DMA,
            pltpu.SemaphoreType.DMA,
            pltpu.SemaphoreType.DMA,
        ],
    )
    def agg_kernel(u_hbm, row_hbm, col_hbm, zr_hbm, raw_hbm,
                   rowv, colv, gv, zv, acc, *sems):
        c = lax.axis_index("c")
        s = lax.axis_index("s")
        gsem = sems[:2]
        ssem = sems[2:]
        pltpu.sync_copy(zr_hbm, zv)

        def pass_body(p, r):
            f = c * CPC + p
            pltpu.sync_copy(zv, acc.at[pl.ds(s * WPS, WPS)])

            @pl.when(s == 0)
            def _():
                pltpu.sync_copy(zv.at[pl.ds(0, 8)], acc.at[pl.ds(THIRD, 8)])

            pltpu.sync_copy(col_hbm.at[r, s], colv)
            pltpu.sync_copy(row_hbm.at[f, s], rowv)
            for k in range(2):
                pltpu.make_async_copy(u_hbm.at[rowv.at[k]], gv.at[k],
                                      gsem[k]).start()
            plsc.subcore_barrier()

            @pl.loop(0, NJ, step=2)
            def _(j):
                for k in range(2):
                    pltpu.make_async_copy(u_hbm.at[rowv.at[j + k]], gv.at[k],
                                          gsem[k]).wait()
                    pltpu.async_copy(gv.at[k], acc.at[colv.at[j + k]],
                                     ssem[k], add=True)

                @pl.when(j + 2 < NJ)
                def _():
                    for k in range(2):
                        pltpu.make_async_copy(gv.at[k], acc.at[colv.at[j + k]],
                                              ssem[k]).wait()
                        pltpu.make_async_copy(u_hbm.at[rowv.at[j + 2 + k]],
                                              gv.at[k], gsem[k]).start()

            for k in range(2):
                pltpu.make_async_copy(gv.at[k], acc.at[colv.at[NJ - 2 + k]],
                                      ssem[k]).wait()

            plsc.subcore_barrier()
            pltpu.sync_copy(acc.at[pl.ds(s * WPS, WPS)],
                            raw_hbm.at[f, pl.ds(r * THIRD + s * WPS, WPS)])
            plsc.subcore_barrier()

        for p in range(CPC):
            for r in range(NR):
                pass_body(p, r)

    return agg_kernel(u_flat, row3, col3r, zeros)


def _vec_spec(d):
    return pl.BlockSpec((1, d), lambda i: (0, 0))


def _chunked(un_ref, un):
    for k in range(NCHUNK):
        un_ref[k] = un[:, k * CH:(k + 1) * CH]


def _tc_prep(x, degp, fg, fb, g0, b0, W0):
    def body(x_ref, deg_ref, fg_ref, fb_ref, g0_ref, b0_ref, w_ref,
             u0_ref, dinv_ref):
        d = deg_ref[0, :, 0:1] + deg_ref[1, :, 0:1] + 1.0
        dinv = lax.rsqrt(jnp.maximum(d, 1.0))
        t0 = g0_ref[...] * ((fg_ref[...] * (x_ref[...] * SINV) + fb_ref[...]) * SINV) + b0_ref[...]
        z0 = jnp.dot(t0, w_ref[...], preferred_element_type=_F32,
                     precision=lax.Precision.HIGHEST)
        _chunked(u0_ref, z0 * dinv)
        dinv_ref[...] = dinv

    return pl.pallas_call(
        body,
        grid=(GRID,),
        in_specs=[
            pl.BlockSpec((NB, IN_DIM), lambda i: (i, 0)),
            pl.BlockSpec((NCORE, NB, CH), lambda i: (0, i, 0)),
            _vec_spec(IN_DIM), _vec_spec(IN_DIM), _vec_spec(IN_DIM), _vec_spec(IN_DIM),
            pl.BlockSpec((IN_DIM, HID), lambda i: (0, 0)),
        ],
        out_specs=[
            pl.BlockSpec((NCHUNK, NB, CH), lambda i: (0, i, 0)),
            pl.BlockSpec((NB, 1), lambda i: (i, 0)),
        ],
        out_shape=[
            jax.ShapeDtypeStruct((NCHUNK, N, CH), _F32),
            jax.ShapeDtypeStruct((N, 1), _F32),
        ],
    )(x, degp, fg, fb, g0, b0, W0)


def _tc_mid(raw, u, hprev, dinv, bias_prev, W, bng, bnb):
    def body(raw_ref, u_ref, hp_ref, dinv_ref, bp_ref, w_ref, g_ref, bb_ref,
             h_ref, un_ref):
        dinv = dinv_ref[...]
        z = jnp.concatenate(
            [dinv * (raw_ref[k] + u_ref[k]) for k in range(NCHUNK)], axis=1)
        h = jnp.maximum(z + bp_ref[...], 0.0) + hp_ref[...]
        h_ref[...] = h
        t = g_ref[...] * (h * SINV) + bb_ref[...]
        zn = jnp.dot(t, w_ref[...], preferred_element_type=_F32,
                     precision=lax.Precision.HIGHEST)
        _chunked(un_ref, zn * dinv)

    return pl.pallas_call(
        body,
        grid=(GRID,),
        in_specs=[
            pl.BlockSpec((NCHUNK, NB, CH), lambda i: (0, i, 0)),
            pl.BlockSpec((NCHUNK, NB, CH), lambda i: (0, i, 0)),
            pl.BlockSpec((NB, HID), lambda i: (i, 0)),
            pl.BlockSpec((NB, 1), lambda i: (i, 0)),
            _vec_spec(HID),
            pl.BlockSpec((HID, HID), lambda i: (0, 0)),
            _vec_spec(HID), _vec_spec(HID),
        ],
        out_specs=[
            pl.BlockSpec((NB, HID), lambda i: (i, 0)),
            pl.BlockSpec((NCHUNK, NB, CH), lambda i: (0, i, 0)),
        ],
        out_shape=[
            jax.ShapeDtypeStruct((N, HID), _F32),
            jax.ShapeDtypeStruct((NCHUNK, N, CH), _F32),
        ],
    )(raw, u, hprev, dinv, bias_prev, W, bng, bnb)


def _tc_last(h4, batch2, fcg, fcb, Wfc, bfc, bhg, bhb, Wcls, bcls):
    def body(h_ref, batch_ref,
             fcg_ref, fcb_ref, wfc_ref, bfc_ref, bhg_ref, bhb_ref,
             wcls_ref, bcls_ref, out_ref, pool_acc, cnt_acc):
        i = pl.program_id(0)

        @pl.when(i == 0)
        def _():
            pool_acc[...] = jnp.zeros_like(pool_acc)
            cnt_acc[...] = jnp.zeros_like(cnt_acc)

        h = h_ref[...]
        gids = lax.broadcasted_iota(jnp.int32, (1, GRAPHS), 1)
        onehot = (batch_ref[...] == gids).astype(_F32)
        dn = (((0,), (0,)), ((), ()))
        pool_acc[...] += lax.dot_general(onehot, h, dn,
                                         preferred_element_type=_F32,
                                         precision=lax.Precision.HIGHEST)
        cnt_acc[...] += lax.dot_general(onehot, jnp.ones((NB, CH), _F32), dn,
                                        preferred_element_type=_F32,
                                        precision=lax.Precision.HIGHEST)

        @pl.when(i == GRID - 1)
        def _():
            cnt = cnt_acc[:, 0:1]
            hp = pool_acc[...] / jnp.maximum(cnt, 1.0)
            h_ = fcg_ref[...] * (hp * SINV) + fcb_ref[...]
            h_ = jnp.dot(h_, wfc_ref[...], preferred_element_type=_F32,
                         precision=lax.Precision.HIGHEST) + bfc_ref[...]
            hp = hp + jnp.maximum(h_, 0.0)
            hh = bhg_ref[...] * (hp * SINV) + bhb_ref[...]
            o = jnp.dot(hh, wcls_ref[...], preferred_element_type=_F32,
                        precision=lax.Precision.HIGHEST) + bcls_ref[...]
            m = jnp.max(o, axis=1, keepdims=True)
            lse = m + jnp.log(jnp.sum(jnp.exp(o - m), axis=1, keepdims=True))
            out_ref[...] = o - lse

    return pl.pallas_call(
        body,
        grid=(GRID,),
        in_specs=[
            pl.BlockSpec((NB, HID), lambda i: (i, 0)),
            pl.BlockSpec((NB, 1), lambda i: (i, 0)),
            _vec_spec(HID), _vec_spec(HID),
            pl.BlockSpec((HID, HID), lambda i: (0, 0)),
            _vec_spec(HID), _vec_spec(HID), _vec_spec(HID),
            pl.BlockSpec((HID, OUT), lambda i: (0, 0)),
            _vec_spec(OUT),
        ],
        out_specs=pl.BlockSpec((GRAPHS, OUT), lambda i: (0, 0)),
        out_shape=jax.ShapeDtypeStruct((GRAPHS, OUT), _F32),
        scratch_shapes=[
            pltpu.VMEM((GRAPHS, HID), _F32),
            pltpu.VMEM((GRAPHS, CH), _F32),
        ],
    )(h4, batch2, fcg, fcb, Wfc, bfc, bhg, bhb, Wcls, bcls)


def kernel(x, edge_index, batch, bn_feat_g, bn_feat_b,
           bn0_g, bn0_b, W0, b0, bn1_g, bn1_b, W1, b1,
           bn2_g, bn2_b, W2, b2, bn3_g, bn3_b, W3, b3,
           bnfc_g, bnfc_b, Wfc, bfc, bnh_g, bnh_b, Wcls, bcls):
    v = lambda a: a.reshape(1, -1)
    row = edge_index[0].astype(jnp.int32)
    col = edge_index[1].astype(jnp.int32)
    pad = EPAD - E
    rowp = jnp.concatenate([row, jnp.zeros((pad,), jnp.int32)])
    colp = jnp.concatenate([col, jnp.full((pad,), N, jnp.int32)])
    chs = [jnp.where((colp >= r * THIRD) & (colp < (r + 1) * THIRD),
                     colp - r * THIRD, THIRD) for r in range(NR)]
    col3r = jnp.stack(chs).reshape(NR, NSUB, NJ, CH)
    col_degr = jnp.stack(chs).reshape(
        NR, NCORE, NSUB, EPAD // (NCORE * NSUB * CH), CH)
    row3 = (rowp[None, :] + (jnp.arange(NCHUNK, dtype=jnp.int32) * N)[:, None]
            ).reshape(NCHUNK, NSUB, NJ, CH)
    zerosw = jnp.zeros((WPS, CH), _F32)
    onesw = jnp.ones((CH, CH), _F32)

    degp = _sc_degree(col_degr, onesw, zerosw)
    u0, dinv = _tc_prep(x, degp, v(bn_feat_g), v(bn_feat_b), v(bn0_g), v(bn0_b), W0)

    wstack = jnp.stack([W1, W2, W3, W3])
    bstack = jnp.stack([v(b0), v(b1), v(b2), v(b3)])
    gstack = jnp.stack([v(bn1_g), v(bn2_g), v(bn3_g), v(bn3_g)])
    bbstack = jnp.stack([v(bn1_b), v(bn2_b), v(bn3_b), v(bn3_b)])

    n_it = jnp.where(jnp.max(batch) > jnp.int32(1 << 30), 5, 4).astype(jnp.int32)

    def stage(state):
        i, hprev, u = state
        raw = _sc_aggregate(u.reshape(NCHUNK * N, CH), row3, col3r, zerosw)
        Wn = lax.dynamic_index_in_dim(wstack, i, keepdims=False)
        bp = lax.dynamic_index_in_dim(bstack, i, keepdims=False)
        gn = lax.dynamic_index_in_dim(gstack, i, keepdims=False)
        bbn = lax.dynamic_index_in_dim(bbstack, i, keepdims=False)
        h, u_next = _tc_mid(raw, u, hprev, dinv, bp, Wn, gn, bbn)
        return (i + 1, h, u_next)

    _, h4, _ = lax.while_loop(lambda st: st[0] < n_it, stage,
                              (jnp.int32(0), jnp.zeros((N, HID), _F32), u0))
    return _tc_last(h4, batch.reshape(N, 1),
                    v(bnfc_g), v(bnfc_b), Wfc, v(bfc),
                    v(bnh_g), v(bnh_b), Wcls, v(bcls))
